# baseline (speedup 1.0000x reference)
import functools

import jax
import jax.numpy as jnp
from jax import lax
from jax.experimental import pallas as pl
from jax.experimental.pallas import tpu as pltpu
from jax.experimental.pallas import tpu_sc as plsc

EMB_DIM = 32
SLAB = 4
SLAB_W = SLAB * EMB_DIM
IDX_CHUNK = 128


@functools.cache
def _build(batch: int, num_slabs: int):
  info = plsc.get_sparse_core_info()
  nc, ns, nl = info.num_cores, info.num_subcores, info.num_lanes
  nw = nc * ns
  b_per_w = batch // nw
  n_chunks = b_per_w // IDX_CHUNK
  groups_per_chunk = IDX_CHUNK // nl
  mesh = plsc.VectorSubcoreMesh(core_axis_name="c", subcore_axis_name="s")

  @functools.partial(
      pl.kernel,
      out_type=jax.ShapeDtypeStruct((batch,), jnp.float32),
      mesh=mesh,
      scratch_types=[
          pltpu.VMEM((n_chunks, IDX_CHUNK), jnp.int32),
          pltpu.VMEM((n_chunks, IDX_CHUNK), jnp.int32),
          pltpu.VMEM((n_chunks, IDX_CHUNK), jnp.int32),
          pltpu.VMEM((n_chunks, IDX_CHUNK), jnp.int32),
          pltpu.VMEM((IDX_CHUNK, SLAB_W), jnp.float32),
          pltpu.VMEM((IDX_CHUNK, SLAB_W), jnp.float32),
          pltpu.VMEM((EMB_DIM,), jnp.float32),
          pltpu.VMEM((16,), jnp.float32),
          pltpu.VMEM((b_per_w,), jnp.float32),
          pltpu.SemaphoreType.DMA,
      ],
      compiler_params=pltpu.CompilerParams(
          needs_layout_passes=False, use_tc_tiling_on_sc=True),
  )
  def gmf_kernel(uidx_hbm, iidx_hbm, utab_hbm, itab_hbm, w_hbm, b_hbm,
                 out_hbm, uidx_v, iidx_v, uslab_v, islab_v, uslabs, islabs,
                 w_v, b_v, out_v, sem):
    wid = lax.axis_index("s") * nc + lax.axis_index("c")
    base = wid * b_per_w

    pltpu.sync_copy(uidx_hbm.at[pl.ds(wid * n_chunks, n_chunks)], uidx_v)
    pltpu.sync_copy(iidx_hbm.at[pl.ds(wid * n_chunks, n_chunks)], iidx_v)
    pltpu.sync_copy(w_hbm, w_v)
    pltpu.sync_copy(b_hbm, b_v)

    for k in range(b_per_w // nl):
      j = k // (IDX_CHUNK // nl)
      o = (k % (IDX_CHUNK // nl)) * nl
      uslab_v[j, pl.ds(o, nl)] = lax.shift_right_logical(
          uidx_v[j, pl.ds(o, nl)], 2)
      islab_v[j, pl.ds(o, nl)] = lax.shift_right_logical(
          iidx_v[j, pl.ds(o, nl)], 2)

    bias16 = b_v[...]
    wregs = [w_v[pl.ds(0, nl)], w_v[pl.ds(nl, nl)]]
    lanes = lax.iota(jnp.int32, nl)
    three = jnp.full((nl,), SLAB - 1, jnp.int32)

    for j in range(n_chunks):
      cu = pltpu.async_copy(utab_hbm.at[uslab_v.at[j]], uslabs, sem)
      ci = pltpu.async_copy(itab_hbm.at[islab_v.at[j]], islabs, sem)
      cu.wait()
      ci.wait()

      def group_body(g, _):
        row_ids = g * nl + lanes
        ucol0 = lax.shift_left(
            lax.bitwise_and(uidx_v[j, pl.ds(g * nl, nl)], three), 5)
        icol0 = lax.shift_left(
            lax.bitwise_and(iidx_v[j, pl.ds(g * nl, nl)], three), 5)
        acc = jnp.zeros((nl,), jnp.float32)
        for d in range(EMB_DIM):
          u = plsc.load_gather(uslabs, [row_ids, ucol0 + d])
          it = plsc.load_gather(islabs, [row_ids, icol0 + d])
          acc = acc + u * it * wregs[d // nl][d % nl]
        logits = acc + bias16
        out_v[pl.ds(j * IDX_CHUNK + g * nl, nl)] = (
            1.0 / (1.0 + jnp.exp(-logits)))
        return 0

      lax.fori_loop(0, groups_per_chunk, group_body, 0)

    pltpu.sync_copy(out_v, out_hbm.at[pl.ds(base, b_per_w)])

  return gmf_kernel


def kernel(user_indices, item_indices, embedding_user, embedding_item,
           affine_W, affine_b):
  batch = user_indices.shape[0]
  utp = embedding_user.reshape(-1, SLAB_W)
  itp = embedding_item.reshape(-1, SLAB_W)
  fn = _build(batch, utp.shape[0])
  out = fn(user_indices.astype(jnp.int32).reshape(-1, IDX_CHUNK),
           item_indices.astype(jnp.int32).reshape(-1, IDX_CHUNK),
           utp, itp,
           affine_W.reshape(EMB_DIM),
           jnp.broadcast_to(affine_b.reshape(()), (16,)))
  return out.reshape(batch, 1)


# speedup vs baseline: 3.8194x; 3.8194x over previous
import functools

import jax
import jax.numpy as jnp
from jax import lax
from jax.experimental import pallas as pl
from jax.experimental.pallas import tpu as pltpu
from jax.experimental.pallas import tpu_sc as plsc

EMB_DIM = 32
BLK = 128
HALF = 8


@functools.cache
def _build(batch: int, num_rows: int):
  info = plsc.get_sparse_core_info()
  nc, ns, nl = info.num_cores, info.num_subcores, info.num_lanes
  nw = nc * ns
  b_per_w = batch // nw
  n_groups = b_per_w // nl
  max_off = num_rows - BLK
  mesh = plsc.VectorSubcoreMesh(core_axis_name="c", subcore_axis_name="s")

  @functools.partial(
      pl.kernel,
      out_type=jax.ShapeDtypeStruct((batch,), jnp.float32),
      mesh=mesh,
      scratch_types=[
          pltpu.VMEM((b_per_w,), jnp.int32),
          pltpu.VMEM((b_per_w,), jnp.int32),
          pltpu.VMEM((HALF, EMB_DIM, BLK), jnp.float32),
          pltpu.VMEM((HALF, EMB_DIM, BLK), jnp.float32),
          pltpu.VMEM((EMB_DIM,), jnp.float32),
          pltpu.VMEM((16,), jnp.float32),
          pltpu.VMEM((b_per_w,), jnp.float32),
          pltpu.SemaphoreType.DMA,
      ],
      compiler_params=pltpu.CompilerParams(
          needs_layout_passes=False, use_tc_tiling_on_sc=True),
  )
  def gmf_kernel(uidx_hbm, iidx_hbm, utabT_hbm, itabT_hbm, w_hbm, b_hbm,
                 out_hbm, uidx_v, iidx_v, ublk, iblk, w_v, b_v, out_v, sem):
    wid = lax.axis_index("s") * nc + lax.axis_index("c")
    base = wid * b_per_w

    pltpu.sync_copy(uidx_hbm.at[pl.ds(base, b_per_w)], uidx_v)
    pltpu.sync_copy(iidx_hbm.at[pl.ds(base, b_per_w)], iidx_v)
    pltpu.sync_copy(w_hbm, w_v)
    pltpu.sync_copy(b_hbm, b_v)

    bias16 = b_v[...]
    wregs = [w_v[pl.ds(0, nl)], w_v[pl.ds(nl, nl)]]
    lanes = lax.iota(jnp.int32, nl)
    low7 = jnp.full((nl,), BLK - 1, jnp.int32)
    maxo = jnp.full((nl,), max_off, jnp.int32)
    item_sel = lax.bitwise_and(lanes, jnp.full((nl,), HALF - 1, jnp.int32))
    half_bit = lax.shift_right_logical(lanes, 3)

    def run_half(idx_vec, tab_hbm, blk_ref):
      copies = []
      for m in range(HALF):
        off = pl.multiple_of(
            lax.bitwise_and(idx_vec[m], jnp.int32(~(BLK - 1))), BLK)
        copies.append(pltpu.async_copy(
            tab_hbm.at[:, pl.ds(off, BLK)], blk_ref.at[m], sem))
      lane_vec = lax.bitwise_and(idx_vec, low7)
      return copies, lane_vec

    def group_body(g, _):
      uvec = uidx_v[pl.ds(g * nl, nl)]
      ivec = iidx_v[pl.ds(g * nl, nl)]
      halves = []
      for h in range(2):
        uh = jnp.take(uvec, item_sel + h * HALF)
        ih = jnp.take(ivec, item_sel + h * HALF)
        ucopies, ulane = run_half(uh, utabT_hbm, ublk)
        icopies, ilane = run_half(ih, itabT_hbm, iblk)
        for c in ucopies + icopies:
          c.wait()
        acc = jnp.zeros((nl,), jnp.float32)
        for dp in range(EMB_DIM // 2):
          dvec = 2 * dp + half_bit
          u = plsc.load_gather(ublk, [item_sel, dvec, ulane])
          it = plsc.load_gather(iblk, [item_sel, dvec, ilane])
          we = wregs[(2 * dp) // nl][(2 * dp) % nl]
          wo = wregs[(2 * dp + 1) // nl][(2 * dp + 1) % nl]
          wpair = jnp.where(half_bit == 0, we, wo)
          acc = acc + u * it * wpair
        folded = acc + jnp.take(acc, lax.bitwise_xor(
            lanes, jnp.full((nl,), HALF, jnp.int32)))
        halves.append(folded)
      merged = jnp.where(half_bit == 0, halves[0],
                         jnp.take(halves[1], item_sel))
      logits = merged + bias16
      out_v[pl.ds(g * nl, nl)] = 1.0 / (1.0 + jnp.exp(-logits))
      return 0

    lax.fori_loop(0, n_groups, group_body, 0)

    pltpu.sync_copy(out_v, out_hbm.at[pl.ds(base, b_per_w)])

  return gmf_kernel


def kernel(user_indices, item_indices, embedding_user, embedding_item,
           affine_W, affine_b):
  batch = user_indices.shape[0]
  fn = _build(batch, embedding_user.shape[0])
  out = fn(user_indices.astype(jnp.int32),
           item_indices.astype(jnp.int32),
           embedding_user.T, embedding_item.T,
           affine_W.reshape(EMB_DIM),
           jnp.broadcast_to(affine_b.reshape(()), (16,)))
  return out.reshape(batch, 1)
